# baseline (speedup 1.0000x reference)
import functools

import jax
import jax.numpy as jnp
from jax import lax
from jax.experimental import pallas as pl
from jax.experimental.pallas import tpu as pltpu
from jax.experimental.pallas import tpu_sc as plsc

_BLK = 256
_K = 1024
_D = 256
_BT = 32768
_KT = 128

_NC = 2
_NS = 16
_NW = _NC * _NS
_B_PER_W = _BT // _NW
_CHUNK = 128
_NCHUNK = _B_PER_W // _CHUNK


def _c2_once(cb_ref, c2_ref):
    cb = cb_ref[...]
    c2_ref[...] = jnp.sum(cb * cb, axis=1)[None, :]


def _argmin_block(flat_ref, cb_ref, c2_ref, idx_ref):
    x = flat_ref[...]
    cb = cb_ref[...]
    c2 = c2_ref[...]
    s2 = lax.dot_general(
        x * -2.0, cb, (((1,), (1,)), ((), ())),
        preferred_element_type=jnp.float32)
    x2 = jnp.sum(x * x, axis=1, keepdims=True)

    lane = lax.broadcasted_iota(
        jnp.int32, (_BLK, _KT), 1).astype(jnp.float32)
    m = None
    im = None
    for t in range(_K // _KT):
        d_t = ((x2 + s2[:, t * _KT:(t + 1) * _KT])
               + c2[:, t * _KT:(t + 1) * _KT])
        if t == 0:
            m = d_t
            im = lane
        else:
            better = d_t < m
            im = jnp.where(better, lane + float(t * _KT), im)
            m = jnp.minimum(d_t, m)
    dmin = jnp.min(m, axis=1, keepdims=True)
    idx = jnp.min(jnp.where(m == dmin, im, float(_K)),
                  axis=1, keepdims=True)
    idx_ref[...] = idx.astype(jnp.int32)


def _compute_indices(flat, codebook):
    c2 = pl.pallas_call(
        _c2_once,
        out_shape=jax.ShapeDtypeStruct((1, _K), jnp.float32),
    )(codebook)
    nblk = _BT // _BLK
    idx_col = pl.pallas_call(
        _argmin_block,
        grid=(nblk,),
        in_specs=[
            pl.BlockSpec((_BLK, _D), lambda i: (i, 0)),
            pl.BlockSpec((_K, _D), lambda i: (0, 0)),
            pl.BlockSpec((1, _K), lambda i: (0, 0)),
        ],
        out_specs=pl.BlockSpec((_BLK, 1), lambda i: (i, 0)),
        out_shape=jax.ShapeDtypeStruct((_BT, 1), jnp.int32),
        compiler_params=pltpu.CompilerParams(
            dimension_semantics=("parallel",)),
    )(flat, codebook, c2)
    return idx_col.reshape(_BT)


@functools.partial(
    pl.kernel,
    mesh=plsc.VectorSubcoreMesh(core_axis_name="c", subcore_axis_name="s"),
    out_type=jax.ShapeDtypeStruct((_BT, _D), jnp.float32),
    scratch_types=[
        pltpu.VMEM((2, _CHUNK), jnp.int32),
        pltpu.VMEM((2, _CHUNK, _D), jnp.float32),
        pltpu.SemaphoreType.DMA((2,)),
        pltpu.SemaphoreType.DMA((2,)),
    ],
)
def _sc_gather(table_hbm, idx_hbm, out_hbm, idx_v, rows_v, gsem, wsem):
    wid = lax.axis_index("s") * _NC + lax.axis_index("c")
    base = wid * _B_PER_W
    gathers = [None] * _NCHUNK
    writes = [None] * _NCHUNK
    for j in range(_NCHUNK):
        bf = j % 2
        if j >= 2:
            writes[j - 2].wait()
        off = base + j * _CHUNK
        pltpu.sync_copy(idx_hbm.at[pl.ds(off, _CHUNK)], idx_v.at[bf])
        gathers[j] = pltpu.async_copy(
            table_hbm.at[idx_v.at[bf]], rows_v.at[bf], gsem.at[bf])
        if j >= 1:
            pb = (j - 1) % 2
            gathers[j - 1].wait()
            writes[j - 1] = pltpu.async_copy(
                rows_v.at[pb], out_hbm.at[pl.ds(off - _CHUNK, _CHUNK)],
                wsem.at[pb])
    lastb = (_NCHUNK - 1) % 2
    gathers[_NCHUNK - 1].wait()
    writes[_NCHUNK - 1] = pltpu.async_copy(
        rows_v.at[lastb],
        out_hbm.at[pl.ds(base + (_NCHUNK - 1) * _CHUNK, _CHUNK)],
        wsem.at[lastb])
    writes[_NCHUNK - 2].wait()
    writes[_NCHUNK - 1].wait()


@jax.jit
def kernel(h, codebook):
    b, t, d = h.shape
    flat = h.reshape(_BT, d)
    idx = _compute_indices(flat, codebook)
    q = _sc_gather(codebook, idx)
    return q.reshape(b, t, d), idx.reshape(b, t)


# speedup vs baseline: 1.5387x; 1.5387x over previous
import functools

import jax
import jax.numpy as jnp
from jax import lax
from jax.experimental import pallas as pl
from jax.experimental.pallas import tpu as pltpu
from jax.experimental.pallas import tpu_sc as plsc

_BLK = 1024
_K = 1024
_D = 256
_BT = 32768
_KT = 128

_NC = 2
_NS = 16
_NW = _NC * _NS
_B_PER_W = _BT // _NW
_CHUNK = 128
_NCHUNK = _B_PER_W // _CHUNK


def _c2_once(cb_ref, c2_ref):
    cb = cb_ref[...]
    c2_ref[...] = jnp.sum(cb * cb, axis=1)[None, :]


def _argmin_block(flat_ref, cb_ref, c2_ref, idx_ref):
    x = flat_ref[...]
    cb = cb_ref[...]
    c2 = c2_ref[...]
    s2 = lax.dot_general(
        x * -2.0, cb, (((1,), (1,)), ((), ())),
        preferred_element_type=jnp.float32)
    x2 = jnp.sum(x * x, axis=1, keepdims=True)

    lane = lax.broadcasted_iota(
        jnp.int32, (_BLK, _KT), 1).astype(jnp.float32)
    m = None
    im = None
    for t in range(_K // _KT):
        d_t = ((x2 + s2[:, t * _KT:(t + 1) * _KT])
               + c2[:, t * _KT:(t + 1) * _KT])
        if t == 0:
            m = d_t
            im = lane
        else:
            better = d_t < m
            im = jnp.where(better, lane + float(t * _KT), im)
            m = jnp.minimum(d_t, m)
    dmin = jnp.min(m, axis=1, keepdims=True)
    idx = jnp.min(jnp.where(m == dmin, im, float(_K)),
                  axis=1, keepdims=True)
    idx_ref[...] = idx.astype(jnp.int32)


def _compute_indices(flat, codebook):
    c2 = pl.pallas_call(
        _c2_once,
        out_shape=jax.ShapeDtypeStruct((1, _K), jnp.float32),
    )(codebook)
    nblk = _BT // _BLK
    idx_col = pl.pallas_call(
        _argmin_block,
        grid=(nblk,),
        in_specs=[
            pl.BlockSpec((_BLK, _D), lambda i: (i, 0)),
            pl.BlockSpec((_K, _D), lambda i: (0, 0)),
            pl.BlockSpec((1, _K), lambda i: (0, 0)),
        ],
        out_specs=pl.BlockSpec((_BLK, 1), lambda i: (i, 0)),
        out_shape=jax.ShapeDtypeStruct((_BT, 1), jnp.int32),
        compiler_params=pltpu.CompilerParams(
            dimension_semantics=("parallel",)),
    )(flat, codebook, c2)
    return idx_col.reshape(_BT)


@functools.partial(
    pl.kernel,
    mesh=plsc.VectorSubcoreMesh(core_axis_name="c", subcore_axis_name="s"),
    out_type=jax.ShapeDtypeStruct((_BT, _D), jnp.float32),
    scratch_types=[
        pltpu.VMEM((2, _CHUNK), jnp.int32),
        pltpu.VMEM((2, _CHUNK, _D), jnp.float32),
        pltpu.SemaphoreType.DMA((2,)),
        pltpu.SemaphoreType.DMA((2,)),
    ],
)
def _sc_gather(table_hbm, idx_hbm, out_hbm, idx_v, rows_v, gsem, wsem):
    wid = lax.axis_index("s") * _NC + lax.axis_index("c")
    base = wid * _B_PER_W
    gathers = [None] * _NCHUNK
    writes = [None] * _NCHUNK
    for j in range(_NCHUNK):
        bf = j % 2
        if j >= 2:
            writes[j - 2].wait()
        off = base + j * _CHUNK
        pltpu.sync_copy(idx_hbm.at[pl.ds(off, _CHUNK)], idx_v.at[bf])
        gathers[j] = pltpu.async_copy(
            table_hbm.at[idx_v.at[bf]], rows_v.at[bf], gsem.at[bf])
        if j >= 1:
            pb = (j - 1) % 2
            gathers[j - 1].wait()
            writes[j - 1] = pltpu.async_copy(
                rows_v.at[pb], out_hbm.at[pl.ds(off - _CHUNK, _CHUNK)],
                wsem.at[pb])
    lastb = (_NCHUNK - 1) % 2
    gathers[_NCHUNK - 1].wait()
    writes[_NCHUNK - 1] = pltpu.async_copy(
        rows_v.at[lastb],
        out_hbm.at[pl.ds(base + (_NCHUNK - 1) * _CHUNK, _CHUNK)],
        wsem.at[lastb])
    writes[_NCHUNK - 2].wait()
    writes[_NCHUNK - 1].wait()


@jax.jit
def kernel(h, codebook):
    b, t, d = h.shape
    flat = h.reshape(_BT, d)
    idx = _compute_indices(flat, codebook)
    q = _sc_gather(codebook, idx)
    return q.reshape(b, t, d), idx.reshape(b, t)


# speedup vs baseline: 1.6718x; 1.0865x over previous
import functools

import jax
import jax.numpy as jnp
from jax import lax
from jax.experimental import pallas as pl
from jax.experimental.pallas import tpu as pltpu
from jax.experimental.pallas import tpu_sc as plsc

_BLK = 2048
_K = 1024
_D = 256
_BT = 32768
_KT = 128

_NC = 2
_NS = 16
_NW = _NC * _NS
_B_PER_W = _BT // _NW
_CHUNK = 128
_NCHUNK = _B_PER_W // _CHUNK


def _c2_once(cb_ref, c2_ref):
    cb = cb_ref[...]
    c2_ref[...] = jnp.sum(cb * cb, axis=1)[None, :]


def _argmin_block(flat_ref, cb_ref, c2_ref, idx_ref):
    x = flat_ref[...]
    cb = cb_ref[...]
    c2 = c2_ref[...]
    s2 = lax.dot_general(
        x * -2.0, cb, (((1,), (1,)), ((), ())),
        preferred_element_type=jnp.float32)
    x2 = jnp.sum(x * x, axis=1, keepdims=True)

    lane = lax.broadcasted_iota(
        jnp.int32, (_BLK, _KT), 1).astype(jnp.float32)
    m = None
    im = None
    for t in range(_K // _KT):
        d_t = ((x2 + s2[:, t * _KT:(t + 1) * _KT])
               + c2[:, t * _KT:(t + 1) * _KT])
        if t == 0:
            m = d_t
            im = lane
        else:
            better = d_t < m
            im = jnp.where(better, lane + float(t * _KT), im)
            m = jnp.minimum(d_t, m)
    dmin = jnp.min(m, axis=1, keepdims=True)
    idx = jnp.min(jnp.where(m == dmin, im, float(_K)),
                  axis=1, keepdims=True)
    idx_ref[...] = idx.astype(jnp.int32)


def _compute_indices(flat, codebook):
    c2 = pl.pallas_call(
        _c2_once,
        out_shape=jax.ShapeDtypeStruct((1, _K), jnp.float32),
    )(codebook)
    nblk = _BT // _BLK
    idx_col = pl.pallas_call(
        _argmin_block,
        grid=(nblk,),
        in_specs=[
            pl.BlockSpec((_BLK, _D), lambda i: (i, 0)),
            pl.BlockSpec((_K, _D), lambda i: (0, 0)),
            pl.BlockSpec((1, _K), lambda i: (0, 0)),
        ],
        out_specs=pl.BlockSpec((_BLK, 1), lambda i: (i, 0)),
        out_shape=jax.ShapeDtypeStruct((_BT, 1), jnp.int32),
        compiler_params=pltpu.CompilerParams(
            dimension_semantics=("parallel",)),
    )(flat, codebook, c2)
    return idx_col.reshape(_BT)


@functools.partial(
    pl.kernel,
    mesh=plsc.VectorSubcoreMesh(core_axis_name="c", subcore_axis_name="s"),
    out_type=jax.ShapeDtypeStruct((_BT, _D), jnp.float32),
    scratch_types=[
        pltpu.VMEM((2, _CHUNK), jnp.int32),
        pltpu.VMEM((2, _CHUNK, _D), jnp.float32),
        pltpu.SemaphoreType.DMA((2,)),
        pltpu.SemaphoreType.DMA((2,)),
    ],
)
def _sc_gather(table_hbm, idx_hbm, out_hbm, idx_v, rows_v, gsem, wsem):
    wid = lax.axis_index("s") * _NC + lax.axis_index("c")
    base = wid * _B_PER_W
    gathers = [None] * _NCHUNK
    writes = [None] * _NCHUNK
    for j in range(_NCHUNK):
        bf = j % 2
        if j >= 2:
            writes[j - 2].wait()
        off = base + j * _CHUNK
        pltpu.sync_copy(idx_hbm.at[pl.ds(off, _CHUNK)], idx_v.at[bf])
        gathers[j] = pltpu.async_copy(
            table_hbm.at[idx_v.at[bf]], rows_v.at[bf], gsem.at[bf])
        if j >= 1:
            pb = (j - 1) % 2
            gathers[j - 1].wait()
            writes[j - 1] = pltpu.async_copy(
                rows_v.at[pb], out_hbm.at[pl.ds(off - _CHUNK, _CHUNK)],
                wsem.at[pb])
    lastb = (_NCHUNK - 1) % 2
    gathers[_NCHUNK - 1].wait()
    writes[_NCHUNK - 1] = pltpu.async_copy(
        rows_v.at[lastb],
        out_hbm.at[pl.ds(base + (_NCHUNK - 1) * _CHUNK, _CHUNK)],
        wsem.at[lastb])
    writes[_NCHUNK - 2].wait()
    writes[_NCHUNK - 1].wait()


@jax.jit
def kernel(h, codebook):
    b, t, d = h.shape
    flat = h.reshape(_BT, d)
    idx = _compute_indices(flat, codebook)
    q = _sc_gather(codebook, idx)
    return q.reshape(b, t, d), idx.reshape(b, t)


# speedup vs baseline: 1.6833x; 1.0068x over previous
import functools

import jax
import jax.numpy as jnp
from jax import lax
from jax.experimental import pallas as pl
from jax.experimental.pallas import tpu as pltpu
from jax.experimental.pallas import tpu_sc as plsc

_BLK = 4096
_K = 1024
_D = 256
_BT = 32768
_KT = 128

_NC = 2
_NS = 16
_NW = _NC * _NS
_B_PER_W = _BT // _NW
_CHUNK = 128
_NCHUNK = _B_PER_W // _CHUNK


def _c2_once(cb_ref, c2_ref):
    cb = cb_ref[...]
    c2_ref[...] = jnp.sum(cb * cb, axis=1)[None, :]


def _argmin_block(flat_ref, cb_ref, c2_ref, idx_ref):
    x = flat_ref[...]
    cb = cb_ref[...]
    c2 = c2_ref[...]
    s2 = lax.dot_general(
        x * -2.0, cb, (((1,), (1,)), ((), ())),
        preferred_element_type=jnp.float32)
    x2 = jnp.sum(x * x, axis=1, keepdims=True)

    lane = lax.broadcasted_iota(
        jnp.int32, (_BLK, _KT), 1).astype(jnp.float32)
    m = None
    im = None
    for t in range(_K // _KT):
        d_t = ((x2 + s2[:, t * _KT:(t + 1) * _KT])
               + c2[:, t * _KT:(t + 1) * _KT])
        if t == 0:
            m = d_t
            im = lane
        else:
            better = d_t < m
            im = jnp.where(better, lane + float(t * _KT), im)
            m = jnp.minimum(d_t, m)
    dmin = jnp.min(m, axis=1, keepdims=True)
    idx = jnp.min(jnp.where(m == dmin, im, float(_K)),
                  axis=1, keepdims=True)
    idx_ref[...] = idx.astype(jnp.int32)


def _compute_indices(flat, codebook):
    c2 = pl.pallas_call(
        _c2_once,
        out_shape=jax.ShapeDtypeStruct((1, _K), jnp.float32),
    )(codebook)
    nblk = _BT // _BLK
    idx_col = pl.pallas_call(
        _argmin_block,
        grid=(nblk,),
        in_specs=[
            pl.BlockSpec((_BLK, _D), lambda i: (i, 0)),
            pl.BlockSpec((_K, _D), lambda i: (0, 0)),
            pl.BlockSpec((1, _K), lambda i: (0, 0)),
        ],
        out_specs=pl.BlockSpec((_BLK, 1), lambda i: (i, 0)),
        out_shape=jax.ShapeDtypeStruct((_BT, 1), jnp.int32),
        compiler_params=pltpu.CompilerParams(
            dimension_semantics=("parallel",)),
    )(flat, codebook, c2)
    return idx_col.reshape(_BT)


@functools.partial(
    pl.kernel,
    mesh=plsc.VectorSubcoreMesh(core_axis_name="c", subcore_axis_name="s"),
    out_type=jax.ShapeDtypeStruct((_BT, _D), jnp.float32),
    scratch_types=[
        pltpu.VMEM((2, _CHUNK), jnp.int32),
        pltpu.VMEM((2, _CHUNK, _D), jnp.float32),
        pltpu.SemaphoreType.DMA((2,)),
        pltpu.SemaphoreType.DMA((2,)),
    ],
)
def _sc_gather(table_hbm, idx_hbm, out_hbm, idx_v, rows_v, gsem, wsem):
    wid = lax.axis_index("s") * _NC + lax.axis_index("c")
    base = wid * _B_PER_W
    gathers = [None] * _NCHUNK
    writes = [None] * _NCHUNK
    for j in range(_NCHUNK):
        bf = j % 2
        if j >= 2:
            writes[j - 2].wait()
        off = base + j * _CHUNK
        pltpu.sync_copy(idx_hbm.at[pl.ds(off, _CHUNK)], idx_v.at[bf])
        gathers[j] = pltpu.async_copy(
            table_hbm.at[idx_v.at[bf]], rows_v.at[bf], gsem.at[bf])
        if j >= 1:
            pb = (j - 1) % 2
            gathers[j - 1].wait()
            writes[j - 1] = pltpu.async_copy(
                rows_v.at[pb], out_hbm.at[pl.ds(off - _CHUNK, _CHUNK)],
                wsem.at[pb])
    lastb = (_NCHUNK - 1) % 2
    gathers[_NCHUNK - 1].wait()
    writes[_NCHUNK - 1] = pltpu.async_copy(
        rows_v.at[lastb],
        out_hbm.at[pl.ds(base + (_NCHUNK - 1) * _CHUNK, _CHUNK)],
        wsem.at[lastb])
    writes[_NCHUNK - 2].wait()
    writes[_NCHUNK - 1].wait()


@jax.jit
def kernel(h, codebook):
    b, t, d = h.shape
    flat = h.reshape(_BT, d)
    idx = _compute_indices(flat, codebook)
    q = _sc_gather(codebook, idx)
    return q.reshape(b, t, d), idx.reshape(b, t)


# speedup vs baseline: 1.6848x; 1.0009x over previous
import functools

import jax
import jax.numpy as jnp
from jax import lax
from jax.experimental import pallas as pl
from jax.experimental.pallas import tpu as pltpu
from jax.experimental.pallas import tpu_sc as plsc

_BLK = 8192
_K = 1024
_D = 256
_BT = 32768
_KT = 128

_NC = 2
_NS = 16
_NW = _NC * _NS
_B_PER_W = _BT // _NW
_CHUNK = 128
_NCHUNK = _B_PER_W // _CHUNK


def _c2_once(cb_ref, c2_ref):
    cb = cb_ref[...]
    c2_ref[...] = jnp.sum(cb * cb, axis=1)[None, :]


def _argmin_block(flat_ref, cb_ref, c2_ref, idx_ref):
    x = flat_ref[...]
    cb = cb_ref[...]
    c2 = c2_ref[...]
    s2 = lax.dot_general(
        x * -2.0, cb, (((1,), (1,)), ((), ())),
        preferred_element_type=jnp.float32)
    x2 = jnp.sum(x * x, axis=1, keepdims=True)

    lane = lax.broadcasted_iota(
        jnp.int32, (_BLK, _KT), 1).astype(jnp.float32)
    m = None
    im = None
    for t in range(_K // _KT):
        d_t = ((x2 + s2[:, t * _KT:(t + 1) * _KT])
               + c2[:, t * _KT:(t + 1) * _KT])
        if t == 0:
            m = d_t
            im = lane
        else:
            better = d_t < m
            im = jnp.where(better, lane + float(t * _KT), im)
            m = jnp.minimum(d_t, m)
    dmin = jnp.min(m, axis=1, keepdims=True)
    idx = jnp.min(jnp.where(m == dmin, im, float(_K)),
                  axis=1, keepdims=True)
    idx_ref[...] = idx.astype(jnp.int32)


def _compute_indices(flat, codebook):
    c2 = pl.pallas_call(
        _c2_once,
        out_shape=jax.ShapeDtypeStruct((1, _K), jnp.float32),
    )(codebook)
    nblk = _BT // _BLK
    idx_col = pl.pallas_call(
        _argmin_block,
        grid=(nblk,),
        in_specs=[
            pl.BlockSpec((_BLK, _D), lambda i: (i, 0)),
            pl.BlockSpec((_K, _D), lambda i: (0, 0)),
            pl.BlockSpec((1, _K), lambda i: (0, 0)),
        ],
        out_specs=pl.BlockSpec((_BLK, 1), lambda i: (i, 0)),
        out_shape=jax.ShapeDtypeStruct((_BT, 1), jnp.int32),
        compiler_params=pltpu.CompilerParams(
            dimension_semantics=("parallel",)),
    )(flat, codebook, c2)
    return idx_col.reshape(_BT)


@functools.partial(
    pl.kernel,
    mesh=plsc.VectorSubcoreMesh(core_axis_name="c", subcore_axis_name="s"),
    out_type=jax.ShapeDtypeStruct((_BT, _D), jnp.float32),
    scratch_types=[
        pltpu.VMEM((2, _CHUNK), jnp.int32),
        pltpu.VMEM((2, _CHUNK, _D), jnp.float32),
        pltpu.SemaphoreType.DMA((2,)),
        pltpu.SemaphoreType.DMA((2,)),
    ],
)
def _sc_gather(table_hbm, idx_hbm, out_hbm, idx_v, rows_v, gsem, wsem):
    wid = lax.axis_index("s") * _NC + lax.axis_index("c")
    base = wid * _B_PER_W
    gathers = [None] * _NCHUNK
    writes = [None] * _NCHUNK
    for j in range(_NCHUNK):
        bf = j % 2
        if j >= 2:
            writes[j - 2].wait()
        off = base + j * _CHUNK
        pltpu.sync_copy(idx_hbm.at[pl.ds(off, _CHUNK)], idx_v.at[bf])
        gathers[j] = pltpu.async_copy(
            table_hbm.at[idx_v.at[bf]], rows_v.at[bf], gsem.at[bf])
        if j >= 1:
            pb = (j - 1) % 2
            gathers[j - 1].wait()
            writes[j - 1] = pltpu.async_copy(
                rows_v.at[pb], out_hbm.at[pl.ds(off - _CHUNK, _CHUNK)],
                wsem.at[pb])
    lastb = (_NCHUNK - 1) % 2
    gathers[_NCHUNK - 1].wait()
    writes[_NCHUNK - 1] = pltpu.async_copy(
        rows_v.at[lastb],
        out_hbm.at[pl.ds(base + (_NCHUNK - 1) * _CHUNK, _CHUNK)],
        wsem.at[lastb])
    writes[_NCHUNK - 2].wait()
    writes[_NCHUNK - 1].wait()


@jax.jit
def kernel(h, codebook):
    b, t, d = h.shape
    flat = h.reshape(_BT, d)
    idx = _compute_indices(flat, codebook)
    q = _sc_gather(codebook, idx)
    return q.reshape(b, t, d), idx.reshape(b, t)


# speedup vs baseline: 1.6922x; 1.0044x over previous
import functools

import jax
import jax.numpy as jnp
from jax import lax
from jax.experimental import pallas as pl
from jax.experimental.pallas import tpu as pltpu
from jax.experimental.pallas import tpu_sc as plsc

_BLK = 4096
_K = 1024
_D = 256
_BT = 32768
_KT = 128

_NC = 2
_NS = 16
_NW = _NC * _NS
_B_PER_W = _BT // _NW
_CHUNK = 128
_NCHUNK = _B_PER_W // _CHUNK


def _c2_once(cb_ref, c2_ref):
    cb = cb_ref[...]
    c2_ref[...] = jnp.sum(cb * cb, axis=1)[None, :]


def _argmin_block(flat_ref, cb_ref, c2_ref, idx_ref):
    x = flat_ref[...]
    cb = cb_ref[...]
    c2 = c2_ref[...]
    s2 = lax.dot_general(
        x * -2.0, cb, (((1,), (1,)), ((), ())),
        preferred_element_type=jnp.float32)
    x2 = jnp.sum(x * x, axis=1, keepdims=True)

    lane = lax.broadcasted_iota(
        jnp.int32, (_BLK, _KT), 1).astype(jnp.float32)
    m = None
    im = None
    for t in range(_K // _KT):
        d_t = ((x2 + s2[:, t * _KT:(t + 1) * _KT])
               + c2[:, t * _KT:(t + 1) * _KT])
        if t == 0:
            m = d_t
            im = lane
        else:
            better = d_t < m
            im = jnp.where(better, lane + float(t * _KT), im)
            m = jnp.minimum(d_t, m)
    dmin = jnp.min(m, axis=1, keepdims=True)
    idx = jnp.min(jnp.where(m == dmin, im, float(_K)),
                  axis=1, keepdims=True)
    idx_ref[...] = idx.astype(jnp.int32)


def _compute_indices(flat, codebook):
    c2 = pl.pallas_call(
        _c2_once,
        out_shape=jax.ShapeDtypeStruct((1, _K), jnp.float32),
    )(codebook)
    nblk = _BT // _BLK
    idx_col = pl.pallas_call(
        _argmin_block,
        grid=(nblk,),
        in_specs=[
            pl.BlockSpec((_BLK, _D), lambda i: (i, 0)),
            pl.BlockSpec((_K, _D), lambda i: (0, 0)),
            pl.BlockSpec((1, _K), lambda i: (0, 0)),
        ],
        out_specs=pl.BlockSpec((_BLK, 1), lambda i: (i, 0)),
        out_shape=jax.ShapeDtypeStruct((_BT, 1), jnp.int32),
        compiler_params=pltpu.CompilerParams(
            dimension_semantics=("parallel",)),
    )(flat, codebook, c2)
    return idx_col.reshape(_BT)


@functools.partial(
    pl.kernel,
    mesh=plsc.VectorSubcoreMesh(core_axis_name="c", subcore_axis_name="s"),
    out_type=jax.ShapeDtypeStruct((_BT, _D), jnp.float32),
    scratch_types=[
        pltpu.VMEM((2, _CHUNK), jnp.int32),
        pltpu.VMEM((2, _CHUNK, _D), jnp.float32),
        pltpu.SemaphoreType.DMA((2,)),
        pltpu.SemaphoreType.DMA((2,)),
    ],
)
def _sc_gather(table_hbm, idx_hbm, out_hbm, idx_v, rows_v, gsem, wsem):
    wid = lax.axis_index("s") * _NC + lax.axis_index("c")
    base = wid * _B_PER_W
    gathers = [None] * _NCHUNK
    writes = [None] * _NCHUNK
    for j in range(_NCHUNK):
        bf = j % 2
        if j >= 2:
            writes[j - 2].wait()
        off = base + j * _CHUNK
        pltpu.sync_copy(idx_hbm.at[pl.ds(off, _CHUNK)], idx_v.at[bf])
        gathers[j] = pltpu.async_copy(
            table_hbm.at[idx_v.at[bf]], rows_v.at[bf], gsem.at[bf])
        if j >= 1:
            pb = (j - 1) % 2
            gathers[j - 1].wait()
            writes[j - 1] = pltpu.async_copy(
                rows_v.at[pb], out_hbm.at[pl.ds(off - _CHUNK, _CHUNK)],
                wsem.at[pb])
    lastb = (_NCHUNK - 1) % 2
    gathers[_NCHUNK - 1].wait()
    writes[_NCHUNK - 1] = pltpu.async_copy(
        rows_v.at[lastb],
        out_hbm.at[pl.ds(base + (_NCHUNK - 1) * _CHUNK, _CHUNK)],
        wsem.at[lastb])
    writes[_NCHUNK - 2].wait()
    writes[_NCHUNK - 1].wait()


@jax.jit
def kernel(h, codebook):
    b, t, d = h.shape
    flat = h.reshape(_BT, d)
    idx = _compute_indices(flat, codebook)
    q = _sc_gather(codebook, idx)
    return q.reshape(b, t, d), idx.reshape(b, t)
